# Initial kernel scaffold; baseline (speedup 1.0000x reference)
import jax
import jax.numpy as jnp
from jax.experimental import pallas as pl


def kernel(features, edge_index, norm_A, W0, b0, Wc1, Wc2, Wout, bout):
    raise NotImplementedError("write your pallas kernel here")


# speedup vs baseline: 3.6753x; 3.6753x over previous
import functools
import math

import jax
import jax.numpy as jnp
from jax import lax
from jax.experimental import pallas as pl
from jax.experimental.pallas import tpu as pltpu
from jax.experimental.pallas import tpu_sc as plsc

N = 10000
D = 128
E = 320000
ALPHA = 0.5

NC = 2
NS = 16
NW = NC * NS
CH = 128
CPW = 80
EPW = CPW * CH
EPAD = EPW * NW
NP = 10240
RPT = NP // NS
ZR = 128

_mesh = plsc.VectorSubcoreMesh(core_axis_name="c", subcore_axis_name="s")


DH = D // 2


@functools.partial(
    pl.kernel,
    mesh=_mesh,
    compiler_params=pltpu.CompilerParams(
        needs_layout_passes=False, use_tc_tiling_on_sc=False),
    out_type=jax.ShapeDtypeStruct((NC, 2, NP, DH), jnp.float32),
    scratch_types=[
        pltpu.VMEM((CPW, CH), jnp.int32),
        pltpu.VMEM((CPW, CH), jnp.int32),
        pltpu.VMEM((EPW,), jnp.float32),
        pltpu.VMEM((CH, DH), jnp.float32),
        pltpu.VMEM((ZR, DH), jnp.float32),
        pltpu.VMEM_SHARED((NP, DH), jnp.float32),
        pltpu.SemaphoreType.DMA,
    ],
)
def _sc_edge_pass(x0_hbm, x1_hbm, src_hbm, dst_hbm, norm_hbm, out_hbm,
                  src_v, dst_v, norm_v, rows_v, zero_v, hi_sh, sem):
    c = lax.axis_index("c")
    s = lax.axis_index("s")
    w = c * NS + s

    def _zero_row(i, _):
        for f in range(DH // 16):
            zero_v[i, pl.ds(f * 16, 16)] = jnp.zeros((16,), jnp.float32)
        return 0
    lax.fori_loop(0, ZR, _zero_row, 0)

    pltpu.sync_copy(src_hbm.at[pl.ds(w * CPW, CPW)], src_v)
    pltpu.sync_copy(dst_hbm.at[pl.ds(w * CPW, CPW)], dst_v)
    pltpu.sync_copy(norm_hbm.at[pl.ds(w * EPW, EPW)], norm_v)

    for half, xh_hbm in ((0, x0_hbm), (1, x1_hbm)):
        for k in range(RPT // ZR):
            pltpu.sync_copy(zero_v, hi_sh.at[pl.ds(s * RPT + k * ZR, ZR)])

        plsc.subcore_barrier()

        def _chunk(j, _):
            pltpu.async_copy(xh_hbm.at[src_v.at[j]], rows_v, sem).wait()

            jbase = jnp.full((16,), j * CH, jnp.int32)

            def _scale(e, _):
                g = plsc.load_gather(norm_v, [jbase + e])
                for f in range(DH // 16):
                    rows_v[e, pl.ds(f * 16, 16)] = (
                        rows_v[e, pl.ds(f * 16, 16)] * g)
                return 0
            lax.fori_loop(0, CH, _scale, 0)

            pltpu.sync_copy(rows_v, hi_sh.at[dst_v.at[j]], add=True)
            return 0
        lax.fori_loop(0, CPW, _chunk, 0)

        plsc.subcore_barrier()

        pltpu.sync_copy(hi_sh.at[pl.ds(s * RPT, RPT)],
                        out_hbm.at[c, half, pl.ds(s * RPT, RPT)])


_BLK = 2000


def _tc_pre_body(feat_ref, w_ref, b_ref, out_ref):
    z = jnp.dot(feat_ref[...], w_ref[...], preferred_element_type=jnp.float32)
    out_ref[...] = jnp.maximum(z + b_ref[...], 0.0)


def _tc_mid_body(p0_ref, p1_ref, h0_ref, w_ref, x_ref, m_ref):
    hi = p0_ref[...] + p1_ref[...]
    support = (1.0 - ALPHA) * hi + ALPHA * h0_ref[...]
    x = jnp.maximum(
        jnp.dot(support, w_ref[...], preferred_element_type=jnp.float32), 0.0)
    x_ref[...] = x
    m_ref[...] = jnp.maximum(h0_ref[...], x)


def _tc_fin_body(p0_ref, p1_ref, h0_ref, m_ref, w_ref, wo_ref, bo_ref, out_ref):
    hi = p0_ref[...] + p1_ref[...]
    support = (1.0 - ALPHA) * hi + ALPHA * h0_ref[...]
    x = jnp.maximum(
        jnp.dot(support, w_ref[...], preferred_element_type=jnp.float32), 0.0)
    m = jnp.maximum(m_ref[...], x)
    z = jnp.dot(m, wo_ref[...], preferred_element_type=jnp.float32) + bo_ref[...]
    zmax = jnp.max(z, axis=1, keepdims=True)
    lse = jnp.log(jnp.sum(jnp.exp(z - zmax), axis=1, keepdims=True)) + zmax
    out_ref[...] = z - lse


def _row_spec():
    return pl.BlockSpec((_BLK, D), lambda i: (i, 0))


def _full_spec():
    return pl.BlockSpec((D, D), lambda i: (0, 0))


def _bias_spec():
    return pl.BlockSpec((1, D), lambda i: (0, 0))


def _edge_sweep(x, src_p, dst_p, norm_p):
    out = _sc_edge_pass(x[:, :DH], x[:, DH:], src_p, dst_p, norm_p)
    return jnp.concatenate([out[:, 0, :N, :], out[:, 1, :N, :]], axis=-1)


def kernel(features, edge_index, norm_A, W0, b0, Wc1, Wc2, Wout, bout):
    src = edge_index[0]
    dst = edge_index[1]

    pad = EPAD - E
    pad_idx = (jnp.arange(pad, dtype=jnp.int32) * 97) % N
    src_p = jnp.concatenate([src, pad_idx]).reshape(NW * CPW, CH)
    dst_p = jnp.concatenate([dst, pad_idx]).reshape(NW * CPW, CH)
    norm_p = jnp.concatenate([norm_A, jnp.zeros((pad,), jnp.float32)])

    beta1 = math.log(2.0)
    beta2 = math.log(1.5)
    eye = jnp.eye(D, dtype=jnp.float32)
    W1p = beta1 * Wc1 + (1.0 - beta1) * eye
    W2p = beta2 * Wc2 + (1.0 - beta2) * eye
    b0r = b0.reshape(1, D)
    boutr = bout.reshape(1, D)

    grid = (N // _BLK,)

    x = pl.pallas_call(
        _tc_pre_body,
        grid=grid,
        in_specs=[_row_spec(), _full_spec(), _bias_spec()],
        out_specs=_row_spec(),
        out_shape=jax.ShapeDtypeStruct((N, D), jnp.float32),
    )(features, W0, b0r)

    p1 = _edge_sweep(x, src_p, dst_p, norm_p)

    x2, m2 = pl.pallas_call(
        _tc_mid_body,
        grid=grid,
        in_specs=[_row_spec(), _row_spec(), _row_spec(), _full_spec()],
        out_specs=[_row_spec(), _row_spec()],
        out_shape=[jax.ShapeDtypeStruct((N, D), jnp.float32),
                   jax.ShapeDtypeStruct((N, D), jnp.float32)],
    )(p1[0], p1[1], x, W1p)

    p2 = _edge_sweep(x2, src_p, dst_p, norm_p)

    out = pl.pallas_call(
        _tc_fin_body,
        grid=grid,
        in_specs=[_row_spec(), _row_spec(), _row_spec(), _row_spec(),
                  _full_spec(), _full_spec(), _bias_spec()],
        out_specs=_row_spec(),
        out_shape=jax.ShapeDtypeStruct((N, D), jnp.float32),
    )(p2[0], p2[1], x, m2, W2p, Wout, boutr)

    return out


# speedup vs baseline: 5.7517x; 1.5650x over previous
import functools
import math

import jax
import jax.numpy as jnp
from jax import lax
from jax.experimental import pallas as pl
from jax.experimental.pallas import tpu as pltpu
from jax.experimental.pallas import tpu_sc as plsc

N = 10000
D = 128
E = 320000
ALPHA = 0.5

NC = 2
NS = 16
NW = NC * NS
CH = 128
CPW = 80
EPW = CPW * CH
EPAD = EPW * NW
NP = 10240
RPT = NP // NS
ZR = 128

_mesh = plsc.VectorSubcoreMesh(core_axis_name="c", subcore_axis_name="s")


DH = D // 2


@functools.partial(
    pl.kernel,
    mesh=_mesh,
    compiler_params=pltpu.CompilerParams(
        needs_layout_passes=False, use_tc_tiling_on_sc=False),
    out_type=jax.ShapeDtypeStruct((NC, 2, NP, DH), jnp.float32),
    scratch_types=[
        pltpu.VMEM((CPW, CH), jnp.int32),
        pltpu.VMEM((CPW, CH), jnp.int32),
        pltpu.VMEM((EPW,), jnp.float32),
        pltpu.VMEM((CH, DH), jnp.float32),
        pltpu.VMEM((CH, DH), jnp.float32),
        pltpu.VMEM((ZR, DH), jnp.float32),
        pltpu.VMEM_SHARED((NP, DH), jnp.float32),
        pltpu.SemaphoreType.DMA,
        pltpu.SemaphoreType.DMA,
        pltpu.SemaphoreType.DMA,
        pltpu.SemaphoreType.DMA,
    ],
)
def _sc_edge_pass(x0_hbm, x1_hbm, src_hbm, dst_hbm, norm_hbm, out_hbm,
                  src_v, dst_v, norm_v, rows0_v, rows1_v, zero_v, hi_sh,
                  gs0, gs1, ss0, ss1):
    c = lax.axis_index("c")
    s = lax.axis_index("s")
    w = c * NS + s

    def _zero_row(i, _):
        for f in range(DH // 16):
            zero_v[i, pl.ds(f * 16, 16)] = jnp.zeros((16,), jnp.float32)
        return 0
    lax.fori_loop(0, ZR, _zero_row, 0)

    pltpu.sync_copy(src_hbm.at[pl.ds(w * CPW, CPW)], src_v)
    pltpu.sync_copy(dst_hbm.at[pl.ds(w * CPW, CPW)], dst_v)
    pltpu.sync_copy(norm_hbm.at[pl.ds(w * EPW, EPW)], norm_v)

    def _scale_chunk(buf, j):
        jbase = jnp.full((16,), j * CH, jnp.int32)

        def _scale(e, _):
            g = plsc.load_gather(norm_v, [jbase + e])
            for f in range(DH // 16):
                buf[e, pl.ds(f * 16, 16)] = buf[e, pl.ds(f * 16, 16)] * g
            return 0
        lax.fori_loop(0, CH, _scale, 0, unroll=8)

    for half, xh_hbm in ((0, x0_hbm), (1, x1_hbm)):
        for k in range(RPT // ZR):
            pltpu.sync_copy(zero_v, hi_sh.at[pl.ds(s * RPT + k * ZR, ZR)])

        plsc.subcore_barrier()

        pltpu.async_copy(xh_hbm.at[src_v.at[0]], rows0_v, gs0)

        def _pipe(jj, _):
            j0 = 2 * jj
            j1 = j0 + 1

            @pl.when(jj > 0)
            def _():
                pltpu.make_async_copy(rows1_v, hi_sh.at[dst_v.at[j1]],
                                      ss1).wait()
            pltpu.async_copy(xh_hbm.at[src_v.at[j1]], rows1_v, gs1)

            pltpu.make_async_copy(xh_hbm.at[src_v.at[j0]], rows0_v,
                                  gs0).wait()
            _scale_chunk(rows0_v, j0)
            pltpu.async_copy(rows0_v, hi_sh.at[dst_v.at[j0]], ss0, add=True)

            @pl.when(jj + 1 < CPW // 2)
            def _():
                pltpu.make_async_copy(rows0_v, hi_sh.at[dst_v.at[j0]],
                                      ss0).wait()
                pltpu.async_copy(xh_hbm.at[src_v.at[j0 + 2]], rows0_v, gs0)

            pltpu.make_async_copy(xh_hbm.at[src_v.at[j1]], rows1_v,
                                  gs1).wait()
            _scale_chunk(rows1_v, j1)
            pltpu.async_copy(rows1_v, hi_sh.at[dst_v.at[j1]], ss1, add=True)
            return 0
        lax.fori_loop(0, CPW // 2, _pipe, 0)

        pltpu.make_async_copy(rows0_v, hi_sh.at[dst_v.at[0]], ss0).wait()
        pltpu.make_async_copy(rows1_v, hi_sh.at[dst_v.at[0]], ss1).wait()

        plsc.subcore_barrier()

        pltpu.sync_copy(hi_sh.at[pl.ds(s * RPT, RPT)],
                        out_hbm.at[c, half, pl.ds(s * RPT, RPT)])


_BLK = 2000


def _tc_pre_body(feat_ref, w_ref, b_ref, out_ref):
    z = jnp.dot(feat_ref[...], w_ref[...], preferred_element_type=jnp.float32)
    out_ref[...] = jnp.maximum(z + b_ref[...], 0.0)


def _tc_mid_body(p0_ref, p1_ref, h0_ref, w_ref, x_ref, m_ref):
    hi = p0_ref[...] + p1_ref[...]
    support = (1.0 - ALPHA) * hi + ALPHA * h0_ref[...]
    x = jnp.maximum(
        jnp.dot(support, w_ref[...], preferred_element_type=jnp.float32), 0.0)
    x_ref[...] = x
    m_ref[...] = jnp.maximum(h0_ref[...], x)


def _tc_fin_body(p0_ref, p1_ref, h0_ref, m_ref, w_ref, wo_ref, bo_ref, out_ref):
    hi = p0_ref[...] + p1_ref[...]
    support = (1.0 - ALPHA) * hi + ALPHA * h0_ref[...]
    x = jnp.maximum(
        jnp.dot(support, w_ref[...], preferred_element_type=jnp.float32), 0.0)
    m = jnp.maximum(m_ref[...], x)
    z = jnp.dot(m, wo_ref[...], preferred_element_type=jnp.float32) + bo_ref[...]
    zmax = jnp.max(z, axis=1, keepdims=True)
    lse = jnp.log(jnp.sum(jnp.exp(z - zmax), axis=1, keepdims=True)) + zmax
    out_ref[...] = z - lse


def _row_spec():
    return pl.BlockSpec((_BLK, D), lambda i: (i, 0))


def _full_spec():
    return pl.BlockSpec((D, D), lambda i: (0, 0))


def _bias_spec():
    return pl.BlockSpec((1, D), lambda i: (0, 0))


def _edge_sweep(x, src_p, dst_p, norm_p):
    out = _sc_edge_pass(x[:, :DH], x[:, DH:], src_p, dst_p, norm_p)
    return jnp.concatenate([out[:, 0, :N, :], out[:, 1, :N, :]], axis=-1)


def kernel(features, edge_index, norm_A, W0, b0, Wc1, Wc2, Wout, bout):
    src = edge_index[0]
    dst = edge_index[1]

    pad = EPAD - E
    pad_idx = (jnp.arange(pad, dtype=jnp.int32) * 97) % N
    src_p = jnp.concatenate([src, pad_idx]).reshape(NW * CPW, CH)
    dst_p = jnp.concatenate([dst, pad_idx]).reshape(NW * CPW, CH)
    norm_p = jnp.concatenate([norm_A, jnp.zeros((pad,), jnp.float32)])

    beta1 = math.log(2.0)
    beta2 = math.log(1.5)
    eye = jnp.eye(D, dtype=jnp.float32)
    W1p = beta1 * Wc1 + (1.0 - beta1) * eye
    W2p = beta2 * Wc2 + (1.0 - beta2) * eye
    b0r = b0.reshape(1, D)
    boutr = bout.reshape(1, D)

    grid = (N // _BLK,)

    x = pl.pallas_call(
        _tc_pre_body,
        grid=grid,
        in_specs=[_row_spec(), _full_spec(), _bias_spec()],
        out_specs=_row_spec(),
        out_shape=jax.ShapeDtypeStruct((N, D), jnp.float32),
    )(features, W0, b0r)

    p1 = _edge_sweep(x, src_p, dst_p, norm_p)

    x2, m2 = pl.pallas_call(
        _tc_mid_body,
        grid=grid,
        in_specs=[_row_spec(), _row_spec(), _row_spec(), _full_spec()],
        out_specs=[_row_spec(), _row_spec()],
        out_shape=[jax.ShapeDtypeStruct((N, D), jnp.float32),
                   jax.ShapeDtypeStruct((N, D), jnp.float32)],
    )(p1[0], p1[1], x, W1p)

    p2 = _edge_sweep(x2, src_p, dst_p, norm_p)

    out = pl.pallas_call(
        _tc_fin_body,
        grid=grid,
        in_specs=[_row_spec(), _row_spec(), _row_spec(), _row_spec(),
                  _full_spec(), _full_spec(), _bias_spec()],
        out_specs=_row_spec(),
        out_shape=jax.ShapeDtypeStruct((N, D), jnp.float32),
    )(p2[0], p2[1], x, m2, W2p, Wout, boutr)

    return out


# speedup vs baseline: 7.0231x; 1.2210x over previous
import functools
import math

import jax
import jax.numpy as jnp
from jax import lax
from jax.experimental import pallas as pl
from jax.experimental.pallas import tpu as pltpu
from jax.experimental.pallas import tpu_sc as plsc

N = 10000
D = 128
E = 320000
ALPHA = 0.5

NC = 2
NS = 16
NW = NC * NS
CH = 128
CPW = 80
EPW = CPW * CH
EPAD = EPW * NW
NP = 10240
RPT = NP // NS
ZR = 128

_mesh = plsc.VectorSubcoreMesh(core_axis_name="c", subcore_axis_name="s")


DH = D // 2


@functools.partial(
    pl.kernel,
    mesh=_mesh,
    compiler_params=pltpu.CompilerParams(
        needs_layout_passes=False, use_tc_tiling_on_sc=False),
    out_type=jax.ShapeDtypeStruct((NC, 2, NP, DH), jnp.float32),
    scratch_types=[
        pltpu.VMEM((CPW, CH), jnp.int32),
        pltpu.VMEM((CPW, CH), jnp.int32),
        pltpu.VMEM((EPW,), jnp.float32),
        pltpu.VMEM((CH, DH), jnp.float32),
        pltpu.VMEM((CH, DH), jnp.float32),
        pltpu.VMEM((CH, DH), jnp.float32),
        pltpu.VMEM((CH, DH), jnp.float32),
        pltpu.VMEM((ZR, DH), jnp.float32),
        pltpu.VMEM_SHARED((NP, DH), jnp.float32),
        pltpu.SemaphoreType.DMA,
        pltpu.SemaphoreType.DMA,
        pltpu.SemaphoreType.DMA,
        pltpu.SemaphoreType.DMA,
        pltpu.SemaphoreType.DMA,
        pltpu.SemaphoreType.DMA,
        pltpu.SemaphoreType.DMA,
        pltpu.SemaphoreType.DMA,
    ],
)
def _sc_edge_pass(x0_hbm, x1_hbm, src_hbm, dst_hbm, norm_hbm, out_hbm,
                  src_v, dst_v, norm_v, rows0_v, rows1_v, rows2_v, rows3_v,
                  zero_v, hi_sh, gs0, gs1, gs2, gs3, ss0, ss1, ss2, ss3):
    c = lax.axis_index("c")
    s = lax.axis_index("s")
    w = c * NS + s

    def _zero_row(i, _):
        for f in range(DH // 16):
            zero_v[i, pl.ds(f * 16, 16)] = jnp.zeros((16,), jnp.float32)
        return 0
    lax.fori_loop(0, ZR, _zero_row, 0)

    pltpu.sync_copy(src_hbm.at[pl.ds(w * CPW, CPW)], src_v)
    pltpu.sync_copy(dst_hbm.at[pl.ds(w * CPW, CPW)], dst_v)
    pltpu.sync_copy(norm_hbm.at[pl.ds(w * EPW, EPW)], norm_v)

    def _scale_chunk(buf, j):
        jbase = jnp.full((16,), j * CH, jnp.int32)

        def _scale(e, _):
            g = plsc.load_gather(norm_v, [jbase + e])
            for f in range(DH // 16):
                buf[e, pl.ds(f * 16, 16)] = buf[e, pl.ds(f * 16, 16)] * g
            return 0
        lax.fori_loop(0, CH, _scale, 0, unroll=8)

    bufs = (rows0_v, rows1_v, rows2_v, rows3_v)
    gsems = (gs0, gs1, gs2, gs3)
    ssems = (ss0, ss1, ss2, ss3)

    for half, xh_hbm in ((0, x0_hbm), (1, x1_hbm)):
        for k in range(RPT // ZR):
            pltpu.sync_copy(zero_v, hi_sh.at[pl.ds(s * RPT + k * ZR, ZR)])

        plsc.subcore_barrier()

        pltpu.async_copy(xh_hbm.at[src_v.at[0]], bufs[0], gsems[0])
        pltpu.async_copy(xh_hbm.at[src_v.at[1]], bufs[1], gsems[1])

        def _pipe(jj, _):
            for b in range(4):
                j = 4 * jj + b
                bp = (b + 2) % 4

                @pl.when(j >= 2)
                def _():
                    pltpu.make_async_copy(bufs[bp], hi_sh.at[dst_v.at[0]],
                                          ssems[bp]).wait()

                @pl.when(j + 2 < CPW)
                def _():
                    pltpu.async_copy(xh_hbm.at[src_v.at[j + 2]], bufs[bp],
                                     gsems[bp])

                pltpu.make_async_copy(xh_hbm.at[src_v.at[j]], bufs[b],
                                      gsems[b]).wait()
                _scale_chunk(bufs[b], j)
                pltpu.async_copy(bufs[b], hi_sh.at[dst_v.at[j]], ssems[b],
                                 add=True)
            return 0
        lax.fori_loop(0, CPW // 4, _pipe, 0)

        pltpu.make_async_copy(bufs[2], hi_sh.at[dst_v.at[0]], ssems[2]).wait()
        pltpu.make_async_copy(bufs[3], hi_sh.at[dst_v.at[0]], ssems[3]).wait()

        plsc.subcore_barrier()

        pltpu.sync_copy(hi_sh.at[pl.ds(s * RPT, RPT)],
                        out_hbm.at[c, half, pl.ds(s * RPT, RPT)])


_BLK = 2000


def _tc_pre_body(feat_ref, w_ref, b_ref, out_ref):
    z = jnp.dot(feat_ref[...], w_ref[...], preferred_element_type=jnp.float32)
    out_ref[...] = jnp.maximum(z + b_ref[...], 0.0)


def _tc_mid_body(p0_ref, p1_ref, h0_ref, w_ref, x_ref, m_ref):
    hi = p0_ref[...] + p1_ref[...]
    support = (1.0 - ALPHA) * hi + ALPHA * h0_ref[...]
    x = jnp.maximum(
        jnp.dot(support, w_ref[...], preferred_element_type=jnp.float32), 0.0)
    x_ref[...] = x
    m_ref[...] = jnp.maximum(h0_ref[...], x)


def _tc_fin_body(p0_ref, p1_ref, h0_ref, m_ref, w_ref, wo_ref, bo_ref, out_ref):
    hi = p0_ref[...] + p1_ref[...]
    support = (1.0 - ALPHA) * hi + ALPHA * h0_ref[...]
    x = jnp.maximum(
        jnp.dot(support, w_ref[...], preferred_element_type=jnp.float32), 0.0)
    m = jnp.maximum(m_ref[...], x)
    z = jnp.dot(m, wo_ref[...], preferred_element_type=jnp.float32) + bo_ref[...]
    zmax = jnp.max(z, axis=1, keepdims=True)
    lse = jnp.log(jnp.sum(jnp.exp(z - zmax), axis=1, keepdims=True)) + zmax
    out_ref[...] = z - lse


def _row_spec():
    return pl.BlockSpec((_BLK, D), lambda i: (i, 0))


def _full_spec():
    return pl.BlockSpec((D, D), lambda i: (0, 0))


def _bias_spec():
    return pl.BlockSpec((1, D), lambda i: (0, 0))


def _edge_sweep(x, src_p, dst_p, norm_p):
    out = _sc_edge_pass(x[:, :DH], x[:, DH:], src_p, dst_p, norm_p)
    return jnp.concatenate([out[:, 0, :N, :], out[:, 1, :N, :]], axis=-1)


def kernel(features, edge_index, norm_A, W0, b0, Wc1, Wc2, Wout, bout):
    src = edge_index[0]
    dst = edge_index[1]

    pad = EPAD - E
    pad_idx = (jnp.arange(pad, dtype=jnp.int32) * 97) % N
    src_p = jnp.concatenate([src, pad_idx]).reshape(NW * CPW, CH)
    dst_p = jnp.concatenate([dst, pad_idx]).reshape(NW * CPW, CH)
    norm_p = jnp.concatenate([norm_A, jnp.zeros((pad,), jnp.float32)])

    beta1 = math.log(2.0)
    beta2 = math.log(1.5)
    eye = jnp.eye(D, dtype=jnp.float32)
    W1p = beta1 * Wc1 + (1.0 - beta1) * eye
    W2p = beta2 * Wc2 + (1.0 - beta2) * eye
    b0r = b0.reshape(1, D)
    boutr = bout.reshape(1, D)

    grid = (N // _BLK,)

    x = pl.pallas_call(
        _tc_pre_body,
        grid=grid,
        in_specs=[_row_spec(), _full_spec(), _bias_spec()],
        out_specs=_row_spec(),
        out_shape=jax.ShapeDtypeStruct((N, D), jnp.float32),
    )(features, W0, b0r)

    p1 = _edge_sweep(x, src_p, dst_p, norm_p)

    x2, m2 = pl.pallas_call(
        _tc_mid_body,
        grid=grid,
        in_specs=[_row_spec(), _row_spec(), _row_spec(), _full_spec()],
        out_specs=[_row_spec(), _row_spec()],
        out_shape=[jax.ShapeDtypeStruct((N, D), jnp.float32),
                   jax.ShapeDtypeStruct((N, D), jnp.float32)],
    )(p1[0], p1[1], x, W1p)

    p2 = _edge_sweep(x2, src_p, dst_p, norm_p)

    out = pl.pallas_call(
        _tc_fin_body,
        grid=grid,
        in_specs=[_row_spec(), _row_spec(), _row_spec(), _row_spec(),
                  _full_spec(), _full_spec(), _bias_spec()],
        out_specs=_row_spec(),
        out_shape=jax.ShapeDtypeStruct((N, D), jnp.float32),
    )(p2[0], p2[1], x, m2, W2p, Wout, boutr)

    return out
